# Initial kernel scaffold; baseline (speedup 1.0000x reference)
import functools

import jax
import jax.numpy as jnp
from jax import lax
from jax.experimental import pallas as pl
from jax.experimental.pallas import tpu as pltpu
from jax.experimental.pallas import tpu_sc as plsc

N = 10000
E = 320000
D = 128
EPS = 1e-5

NC = 2
NS = 16
NW = NC * NS
EDGES_PER_W = E // NW
K = 80
NCHUNK = EDGES_PER_W // K


def _agg_body(x_hbm, row_hbm, col_hbm, zero_hbm, out_hbm,
              row_v, col_v, gbuf, agg_sp):
    c = lax.axis_index("c")
    s = lax.axis_index("s")
    wid = c * NS + s

    @pl.when(s == 0)
    def _():
        pltpu.sync_copy(zero_hbm, agg_sp)

    plsc.subcore_barrier()

    pltpu.sync_copy(row_hbm.at[wid], row_v)
    pltpu.sync_copy(col_hbm.at[wid], col_v)

    def body(j, carry):
        pltpu.sync_copy(x_hbm.at[col_v.at[j]], gbuf)
        pltpu.sync_copy(gbuf, agg_sp.at[row_v.at[j]], add=True)
        return carry

    lax.fori_loop(0, NCHUNK, body, 0)

    plsc.subcore_barrier()

    rows_per_sub = N // NS
    pltpu.sync_copy(agg_sp.at[pl.ds(s * rows_per_sub, rows_per_sub)],
                    out_hbm.at[c, pl.ds(s * rows_per_sub, rows_per_sub)])


def _agg(x, row3, col3, zero):
    mesh = plsc.VectorSubcoreMesh(core_axis_name="c", subcore_axis_name="s")
    return pl.kernel(
        _agg_body,
        out_type=jax.ShapeDtypeStruct((NC, N, D), jnp.float32),
        mesh=mesh,
        scratch_types=[
            pltpu.VMEM((NCHUNK, K), jnp.int32),
            pltpu.VMEM((NCHUNK, K), jnp.int32),
            pltpu.VMEM((K, D), jnp.float32),
            pltpu.VMEM_SHARED((N, D), jnp.float32),
        ],
    )(x, row3, col3, zero)


def _mlp_body(x_ref, agg_ref, w1_ref, b1_ref, w2_ref, b2_ref, g_ref, be_ref,
              o_ref):
    h = x_ref[...] + agg_ref[0] + agg_ref[1]
    h = jnp.dot(h, w1_ref[...], preferred_element_type=jnp.float32)
    h = jnp.maximum(h + b1_ref[...], 0.0)
    h = jnp.dot(h, w2_ref[...], preferred_element_type=jnp.float32)
    h = h + b2_ref[...]
    mean = jnp.mean(h, axis=0, keepdims=True)
    cen = h - mean
    var = jnp.mean(cen * cen, axis=0, keepdims=True)
    o_ref[...] = cen * lax.rsqrt(var + EPS) * g_ref[...] + be_ref[...]


def _mlp(x, agg, w1, b1, w2, b2, g, be):
    return pl.pallas_call(
        _mlp_body,
        out_shape=jax.ShapeDtypeStruct((N, D), jnp.float32),
    )(x, agg, w1, b1.reshape(1, D), w2, b2.reshape(1, D),
      g.reshape(1, D), be.reshape(1, D))


def kernel(x, edge_index,
           W1_0, b1_0, W2_0, b2_0, g_0, be_0,
           W1_1, b1_1, W2_1, b2_1, g_1, be_1,
           W1_2, b1_2, W2_2, b2_2, g_2, be_2):
    row3 = edge_index[0].reshape(NW, NCHUNK, K)
    col3 = edge_index[1].reshape(NW, NCHUNK, K)
    zero = jnp.zeros((N, D), jnp.float32)
    params = [
        (W1_0, b1_0, W2_0, b2_0, g_0, be_0),
        (W1_1, b1_1, W2_1, b2_1, g_1, be_1),
        (W1_2, b1_2, W2_2, b2_2, g_2, be_2),
    ]
    for (w1, b1, w2, b2, g, be) in params:
        agg = _agg(x, row3, col3, zero)
        x = _mlp(x, agg, w1, b1, w2, b2, g, be)
    return x


# speedup vs baseline: 6.5245x; 6.5245x over previous
import functools

import jax
import jax.numpy as jnp
from jax import lax
from jax.experimental import pallas as pl
from jax.experimental.pallas import tpu as pltpu
from jax.experimental.pallas import tpu_sc as plsc

N = 10000
E = 320000
D = 128
EPS = 1e-5

NC = 2
NS = 16
NW = NC * NS
EDGES_PER_W = E // NW
K = 80
NCHUNK = EDGES_PER_W // K


def _agg_body(x_hbm, row_hbm, col_hbm, zero_hbm, out_hbm,
              row_v, col_v, gbuf, agg_sp):
    c = lax.axis_index("c")
    s = lax.axis_index("s")
    wid = c * NS + s

    @pl.when(s == 0)
    def _():
        pltpu.sync_copy(zero_hbm, agg_sp)

    plsc.subcore_barrier()

    pltpu.sync_copy(row_hbm.at[wid], row_v)
    pltpu.sync_copy(col_hbm.at[wid], col_v)

    def body(j, carry):
        pltpu.sync_copy(x_hbm.at[col_v.at[j]], gbuf)
        pltpu.sync_copy(gbuf, agg_sp.at[row_v.at[j]], add=True)
        return carry

    lax.fori_loop(0, NCHUNK, body, 0)

    plsc.subcore_barrier()

    pltpu.sync_copy(agg_sp.at[pl.ds(s * 624, 624)],
                    out_hbm.at[c, pl.ds(s * 624, 624)])

    @pl.when(s == NS - 1)
    def _():
        pltpu.sync_copy(agg_sp.at[pl.ds(NS * 624, N - NS * 624)],
                        out_hbm.at[c, pl.ds(NS * 624, N - NS * 624)])


def _agg(x, row3, col3, zero):
    mesh = plsc.VectorSubcoreMesh(core_axis_name="c", subcore_axis_name="s")
    return pl.kernel(
        _agg_body,
        out_type=jax.ShapeDtypeStruct((NC, N, D), jnp.float32),
        mesh=mesh,
        scratch_types=[
            pltpu.VMEM((NCHUNK, K), jnp.int32),
            pltpu.VMEM((NCHUNK, K), jnp.int32),
            pltpu.VMEM((K, D), jnp.float32),
            pltpu.VMEM_SHARED((N, D), jnp.float32),
        ],
    )(x, row3, col3, zero)


def _mlp_body(x_ref, agg_ref, w1_ref, b1_ref, w2_ref, b2_ref, g_ref, be_ref,
              o_ref):
    h = x_ref[...] + agg_ref[0] + agg_ref[1]
    h = jnp.dot(h, w1_ref[...], preferred_element_type=jnp.float32)
    h = jnp.maximum(h + b1_ref[...], 0.0)
    h = jnp.dot(h, w2_ref[...], preferred_element_type=jnp.float32)
    h = h + b2_ref[...]
    mean = jnp.mean(h, axis=0, keepdims=True)
    cen = h - mean
    var = jnp.mean(cen * cen, axis=0, keepdims=True)
    o_ref[...] = cen * lax.rsqrt(var + EPS) * g_ref[...] + be_ref[...]


def _mlp(x, agg, w1, b1, w2, b2, g, be):
    return pl.pallas_call(
        _mlp_body,
        out_shape=jax.ShapeDtypeStruct((N, D), jnp.float32),
    )(x, agg, w1, b1.reshape(1, D), w2, b2.reshape(1, D),
      g.reshape(1, D), be.reshape(1, D))


def kernel(x, edge_index,
           W1_0, b1_0, W2_0, b2_0, g_0, be_0,
           W1_1, b1_1, W2_1, b2_1, g_1, be_1,
           W1_2, b1_2, W2_2, b2_2, g_2, be_2):
    row3 = edge_index[0].reshape(NW, NCHUNK, K)
    col3 = edge_index[1].reshape(NW, NCHUNK, K)
    zero = jnp.zeros((N, D), jnp.float32)
    params = [
        (W1_0, b1_0, W2_0, b2_0, g_0, be_0),
        (W1_1, b1_1, W2_1, b2_1, g_1, be_1),
        (W1_2, b1_2, W2_2, b2_2, g_2, be_2),
    ]
    for (w1, b1, w2, b2, g, be) in params:
        agg = _agg(x, row3, col3, zero)
        x = _mlp(x, agg, w1, b1, w2, b2, g, be)
    return x
